# baseline (speedup 1.0000x reference)
import functools

import jax
import jax.numpy as jnp
from jax import lax
from jax.experimental import pallas as pl
from jax.experimental.pallas import tpu as pltpu
from jax.experimental.pallas import tpu_sc as plsc

_N = 10000
_NP = 10240
_E = 320000
_EP = 327680
_C = 128
_TRASH = _N
_NW = 32
_EPW = _EP // _NW
_SL = _NP // 16
_F32 = jnp.float32

_mesh = plsc.VectorSubcoreMesh(core_axis_name="c", subcore_axis_name="s")


def _dot(a, b):
    return lax.dot_general(
        a, b, (((1,), (0,)), ((), ())),
        precision=lax.Precision.HIGHEST, preferred_element_type=_F32)


@functools.partial(
    pl.kernel,
    out_type=jax.ShapeDtypeStruct((2, _NP, _C), _F32),
    mesh=_mesh,
    scratch_types=[
        pltpu.VMEM_SHARED((_NP, _C), _F32),
        pltpu.VMEM((128, _C), _F32),
        pltpu.VMEM((4, 128), jnp.int32),
        pltpu.VMEM((4, 128), jnp.int32),
        pltpu.VMEM((4, 128), jnp.int32),
    ],
)
def _deg_kernel(rows_hbm, cols_hbm, zrow_hbm, deg_hbm,
                acc, obuf, rbuf, cbuf, mbuf):
    c = lax.axis_index("c")
    s = lax.axis_index("s")
    wid = c * 16 + s
    ones16 = jnp.ones((16,), _F32)
    for i in range(128):
        for l in range(8):
            obuf[i, pl.ds(l * 16, 16)] = ones16
    pltpu.sync_copy(zrow_hbm, acc.at[pl.ds(s * _SL, _SL)])
    plsc.subcore_barrier()

    @pl.loop(0, _EPW // 512)
    def _(t):
        base = wid * 80 + t * 4
        pltpu.sync_copy(rows_hbm.at[pl.ds(base, 4)], rbuf)
        pltpu.sync_copy(cols_hbm.at[pl.ds(base, 4)], cbuf)
        for j in range(4):
            for l in range(8):
                rv = rbuf[j, pl.ds(l * 16, 16)]
                cv = cbuf[j, pl.ds(l * 16, 16)]
                mbuf[j, pl.ds(l * 16, 16)] = jnp.where(rv == cv, _TRASH, rv)
        for j in range(4):
            pltpu.sync_copy(obuf, acc.at[mbuf.at[j]], add=True)

    plsc.subcore_barrier()
    pltpu.sync_copy(acc.at[pl.ds(s * _SL, _SL)],
                    deg_hbm.at[c].at[pl.ds(s * _SL, _SL)])


_NWIN = 80
_NSTG = _NWIN // 2
_ZROW = 10200


@functools.partial(
    pl.kernel,
    out_type=jax.ShapeDtypeStruct((2, _N, _C), _F32),
    mesh=_mesh,
    scratch_types=[
        pltpu.VMEM_SHARED((_N, _C), _F32),
        pltpu.VMEM((3, 2, 128), jnp.int32),
        pltpu.VMEM((3, 2, 128), jnp.int32),
        pltpu.VMEM((3, 128, _C), _F32),
        pltpu.SemaphoreType.DMA,
        pltpu.SemaphoreType.DMA,
        pltpu.SemaphoreType.DMA,
        pltpu.SemaphoreType.DMA,
        pltpu.SemaphoreType.DMA,
        pltpu.SemaphoreType.DMA,
        pltpu.SemaphoreType.DMA,
        pltpu.SemaphoreType.DMA,
        pltpu.SemaphoreType.DMA,
    ],
)
def _spmm_kernel(table_hbm, cmask_hbm, rows_hbm, zrow_hbm, part_hbm,
                 acc, cbuf, rbuf, gbuf,
                 sg0, sg1, sg2, si0, si1, si2, ss0, ss1, ss2):
    c = lax.axis_index("c")
    s = lax.axis_index("s")
    wid = c * 16 + s
    sg = (sg0, sg1, sg2)
    si = (si0, si1, si2)
    ss = (ss0, ss1, ss2)

    @pl.when(s < 15)
    def _():
        pltpu.sync_copy(zrow_hbm.at[pl.ds(0, 632)], acc.at[pl.ds(s * 632, 632)])

    @pl.when(s == 15)
    def _():
        pltpu.sync_copy(zrow_hbm.at[pl.ds(0, 520)], acc.at[pl.ds(9480, 520)])

    plsc.subcore_barrier()

    def idx_load(p):
        b = p % 3
        base = wid * _NWIN + p * 2
        pltpu.async_copy(cmask_hbm.at[pl.ds(base, 2)], cbuf.at[b], si[b])
        pltpu.async_copy(rows_hbm.at[pl.ds(base, 2)], rbuf.at[b], si[b])

    def idx_wait(p):
        b = p % 3
        pltpu.make_async_copy(cmask_hbm.at[pl.ds(0, 2)], cbuf.at[b], si[b]).wait()
        pltpu.make_async_copy(rows_hbm.at[pl.ds(0, 2)], rbuf.at[b], si[b]).wait()

    def g_fire(w):
        b = w % 3
        p, k = divmod(w, 2)
        pltpu.async_copy(table_hbm.at[cbuf.at[p % 3, k]], gbuf.at[b], sg[b])

    def g_wait(w):
        b = w % 3
        pltpu.make_async_copy(
            table_hbm.at[cbuf.at[0, 0]], gbuf.at[b], sg[b]).wait()

    def s_fire(w):
        b = w % 3
        p, k = divmod(w, 2)
        pltpu.async_copy(gbuf.at[b], acc.at[rbuf.at[p % 3, k]], ss[b],
                         add=True)

    def s_wait(w):
        b = w % 3
        pltpu.make_async_copy(gbuf.at[b], acc.at[rbuf.at[0, 0]], ss[b]).wait()

    idx_load(0)
    idx_wait(0)
    idx_load(1)
    for w in range(_NWIN):
        p, k = divmod(w, 2)
        if w >= 3:
            s_wait(w - 3)
        if k == 0 and p >= 1:
            idx_wait(p)
            if p + 1 < _NSTG:
                idx_load(p + 1)
        g_fire(w)
        if w >= 1:
            g_wait(w - 1)
            s_fire(w - 1)
    g_wait(_NWIN - 1)
    s_fire(_NWIN - 1)
    for w in range(_NWIN - 3, _NWIN):
        s_wait(w)

    plsc.subcore_barrier()

    @pl.when(s < 15)
    def _():
        pltpu.sync_copy(acc.at[pl.ds(s * 632, 632)],
                        part_hbm.at[c].at[pl.ds(s * 632, 632)])

    @pl.when(s == 15)
    def _():
        pltpu.sync_copy(acc.at[pl.ds(9480, 520)],
                        part_hbm.at[c].at[pl.ds(9480, 520)])


def _tc_out0(xp, w0, w2, b2d, rows, cols):
    def body(x_ref, w0_ref, w2_ref, b_ref, r_ref, c_ref, o_ref, m_ref):
        o_ref[...] = _dot(x_ref[...], w0_ref[...] - w2_ref[...]) + b_ref[...]
        cv = c_ref[...]
        m_ref[...] = jnp.where(r_ref[...] == cv, _ZROW, cv)

    return pl.pallas_call(
        body,
        grid=(10,),
        in_specs=[
            pl.BlockSpec((1024, _C), lambda i: (i, 0)),
            pl.BlockSpec((_C, _C), lambda i: (0, 0)),
            pl.BlockSpec((_C, _C), lambda i: (0, 0)),
            pl.BlockSpec((1, _C), lambda i: (0, 0)),
            pl.BlockSpec((256, 128), lambda i: (i, 0)),
            pl.BlockSpec((256, 128), lambda i: (i, 0)),
        ],
        out_specs=[
            pl.BlockSpec((1024, _C), lambda i: (i, 0)),
            pl.BlockSpec((256, 128), lambda i: (i, 0)),
        ],
        out_shape=[
            jax.ShapeDtypeStruct((_NP, _C), _F32),
            jax.ShapeDtypeStruct((_EP // 128, 128), jnp.int32),
        ],
    )(xp, w0, w2, b2d, rows, cols)


def _tc_scale1(degp, xp):
    def body(d_ref, x_ref, dinv_ref, xs_ref):
        deg = d_ref[0, :, 0:1] + d_ref[1, :, 0:1]
        dinv = jnp.where(deg > 0.0, lax.rsqrt(deg), 0.0)
        dinv_ref[...] = dinv
        xs_ref[...] = dinv * x_ref[...]

    return pl.pallas_call(
        body,
        grid=(10,),
        in_specs=[
            pl.BlockSpec((2, 1024, _C), lambda i: (0, i, 0)),
            pl.BlockSpec((1024, _C), lambda i: (i, 0)),
        ],
        out_specs=[
            pl.BlockSpec((1024, 1), lambda i: (i, 0)),
            pl.BlockSpec((1024, _C), lambda i: (i, 0)),
        ],
        out_shape=[
            jax.ShapeDtypeStruct((_NP, 1), _F32),
            jax.ShapeDtypeStruct((_NP, _C), _F32),
        ],
    )(degp, xp)


def _tc_comb1(part, dinv2, out0, w1):
    def body(p_ref, d_ref, o0_ref, w_ref, o1_ref, y_ref):
        u = p_ref[0] + p_ref[1]
        d = d_ref[...]
        du = d * u
        o1_ref[...] = o0_ref[...] - _dot(du, w_ref[...])
        grow = pl.program_id(0) * 1024 + lax.broadcasted_iota(
            jnp.int32, (1024, 1), 0)
        y_ref[...] = jnp.where(grow < _N, d * du, 0.0)

    return pl.pallas_call(
        body,
        grid=(10,),
        in_specs=[
            pl.BlockSpec((2, 1024, _C), lambda i: (0, i, 0)),
            pl.BlockSpec((1024, 1), lambda i: (i, 0)),
            pl.BlockSpec((1024, _C), lambda i: (i, 0)),
            pl.BlockSpec((_C, _C), lambda i: (0, 0)),
        ],
        out_specs=[
            pl.BlockSpec((1024, _C), lambda i: (i, 0)),
            pl.BlockSpec((1024, _C), lambda i: (i, 0)),
        ],
        out_shape=[
            jax.ShapeDtypeStruct((_NP, _C), _F32),
            jax.ShapeDtypeStruct((_NP, _C), _F32),
        ],
    )(part, dinv2, out0, w1)


def _tc_comb2(part, dinv2, out1, w2):
    def body(p_ref, d_ref, o1_ref, w_ref, o_ref):
        u = p_ref[0] + p_ref[1]
        du = d_ref[...] * u
        o_ref[...] = o1_ref[...] + 2.0 * _dot(du, w_ref[...])

    return pl.pallas_call(
        body,
        grid=(10,),
        in_specs=[
            pl.BlockSpec((2, 1000, _C), lambda i: (0, i, 0)),
            pl.BlockSpec((1000, 1), lambda i: (i, 0)),
            pl.BlockSpec((1000, _C), lambda i: (i, 0)),
            pl.BlockSpec((_C, _C), lambda i: (0, 0)),
        ],
        out_specs=pl.BlockSpec((1000, _C), lambda i: (i, 0)),
        out_shape=jax.ShapeDtypeStruct((_N, _C), _F32),
    )(part, dinv2, out1, w2)


def kernel(x, edge_index, weight, bias):
    xp = jnp.pad(x[0], ((0, _NP - _N), (0, 0)))
    ei = edge_index.astype(jnp.int32)
    rows = jnp.pad(ei[0], (0, _EP - _E)).reshape(_EP // 128, 128)
    cols = jnp.pad(ei[1], (0, _EP - _E)).reshape(_EP // 128, 128)
    zrow = jnp.zeros((_SL, _C), _F32)
    b2d = bias.reshape(1, _C)

    degp = _deg_kernel(rows, cols, zrow)
    out0, cmask = _tc_out0(xp, weight[0], weight[2], b2d, rows, cols)
    dinv2, xs = _tc_scale1(degp, xp)
    part1 = _spmm_kernel(xs, cmask, rows, zrow)
    out1, ytab = _tc_comb1(part1, dinv2, out0, weight[1])
    part2 = _spmm_kernel(ytab, cmask, rows, zrow)
    out = _tc_comb2(part2, dinv2, out1, weight[2])
    return out[None]


# speedup vs baseline: 1.0035x; 1.0035x over previous
import functools

import jax
import jax.numpy as jnp
from jax import lax
from jax.experimental import pallas as pl
from jax.experimental.pallas import tpu as pltpu
from jax.experimental.pallas import tpu_sc as plsc

_N = 10000
_NP = 10240
_E = 320000
_EP = 327680
_C = 128
_TRASH = _N
_NW = 32
_EPW = _EP // _NW
_SL = _NP // 16
_F32 = jnp.float32

_mesh = plsc.VectorSubcoreMesh(core_axis_name="c", subcore_axis_name="s")


def _dot(a, b):
    return lax.dot_general(
        a, b, (((1,), (0,)), ((), ())),
        precision=lax.Precision.HIGHEST, preferred_element_type=_F32)


@functools.partial(
    pl.kernel,
    out_type=jax.ShapeDtypeStruct((2, _NP, _C), _F32),
    mesh=_mesh,
    scratch_types=[
        pltpu.VMEM_SHARED((_NP, _C), _F32),
        pltpu.VMEM((128, _C), _F32),
        pltpu.VMEM((4, 128), jnp.int32),
        pltpu.VMEM((4, 128), jnp.int32),
        pltpu.VMEM((4, 128), jnp.int32),
    ],
)
def _deg_kernel(rows_hbm, cols_hbm, zrow_hbm, deg_hbm,
                acc, obuf, rbuf, cbuf, mbuf):
    c = lax.axis_index("c")
    s = lax.axis_index("s")
    wid = c * 16 + s
    ones16 = jnp.ones((16,), _F32)
    for i in range(128):
        for l in range(8):
            obuf[i, pl.ds(l * 16, 16)] = ones16
    pltpu.sync_copy(zrow_hbm, acc.at[pl.ds(s * _SL, _SL)])
    plsc.subcore_barrier()

    @pl.loop(0, _EPW // 512)
    def _(t):
        base = wid * 80 + t * 4
        pltpu.sync_copy(rows_hbm.at[pl.ds(base, 4)], rbuf)
        pltpu.sync_copy(cols_hbm.at[pl.ds(base, 4)], cbuf)
        for j in range(4):
            for l in range(8):
                rv = rbuf[j, pl.ds(l * 16, 16)]
                cv = cbuf[j, pl.ds(l * 16, 16)]
                mbuf[j, pl.ds(l * 16, 16)] = jnp.where(rv == cv, _TRASH, rv)
        for j in range(4):
            pltpu.sync_copy(obuf, acc.at[mbuf.at[j]], add=True)

    plsc.subcore_barrier()
    pltpu.sync_copy(acc.at[pl.ds(s * _SL, _SL)],
                    deg_hbm.at[c].at[pl.ds(s * _SL, _SL)])


_NWIN = 160
_NSTG = _NWIN // 4
_ZROW = 10200


@functools.partial(
    pl.kernel,
    out_type=jax.ShapeDtypeStruct((2, _N, _C), _F32),
    mesh=_mesh,
    scratch_types=[
        pltpu.VMEM_SHARED((_N, _C), _F32),
        pltpu.VMEM((3, 2, 128), jnp.int32),
        pltpu.VMEM((3, 2, 128), jnp.int32),
        pltpu.VMEM((6, 64, _C), _F32),
        pltpu.SemaphoreType.DMA,
        pltpu.SemaphoreType.DMA,
        pltpu.SemaphoreType.DMA,
        pltpu.SemaphoreType.DMA,
        pltpu.SemaphoreType.DMA,
        pltpu.SemaphoreType.DMA,
        pltpu.SemaphoreType.DMA,
        pltpu.SemaphoreType.DMA,
        pltpu.SemaphoreType.DMA,
        pltpu.SemaphoreType.DMA,
        pltpu.SemaphoreType.DMA,
        pltpu.SemaphoreType.DMA,
        pltpu.SemaphoreType.DMA,
        pltpu.SemaphoreType.DMA,
        pltpu.SemaphoreType.DMA,
    ],
)
def _spmm_kernel(table_hbm, cmask_hbm, rows_hbm, zrow_hbm, part_hbm,
                 acc, cbuf, rbuf, gbuf,
                 sg0, sg1, sg2, sg3, sg4, sg5, si0, si1, si2,
                 ss0, ss1, ss2, ss3, ss4, ss5):
    c = lax.axis_index("c")
    s = lax.axis_index("s")
    wid = c * 16 + s
    sg = (sg0, sg1, sg2, sg3, sg4, sg5)
    si = (si0, si1, si2)
    ss = (ss0, ss1, ss2, ss3, ss4, ss5)

    @pl.when(s < 15)
    def _():
        pltpu.sync_copy(zrow_hbm.at[pl.ds(0, 632)], acc.at[pl.ds(s * 632, 632)])

    @pl.when(s == 15)
    def _():
        pltpu.sync_copy(zrow_hbm.at[pl.ds(0, 520)], acc.at[pl.ds(9480, 520)])

    plsc.subcore_barrier()

    def idx_load(p):
        b = p % 3
        base = wid * (_NWIN // 2) + p * 2
        pltpu.async_copy(cmask_hbm.at[pl.ds(base, 2)], cbuf.at[b], si[b])
        pltpu.async_copy(rows_hbm.at[pl.ds(base, 2)], rbuf.at[b], si[b])

    def idx_wait(p):
        b = p % 3
        pltpu.make_async_copy(cmask_hbm.at[pl.ds(0, 2)], cbuf.at[b], si[b]).wait()
        pltpu.make_async_copy(rows_hbm.at[pl.ds(0, 2)], rbuf.at[b], si[b]).wait()

    def g_fire(w):
        b = w % 6
        p, k = divmod(w, 4)
        r, h = divmod(k, 2)
        pltpu.async_copy(table_hbm.at[cbuf.at[p % 3, r, pl.ds(h * 64, 64)]],
                         gbuf.at[b], sg[b])

    def g_wait(w):
        b = w % 6
        pltpu.make_async_copy(
            table_hbm.at[cbuf.at[0, 0, pl.ds(0, 64)]], gbuf.at[b],
            sg[b]).wait()

    def s_fire(w):
        b = w % 6
        p, k = divmod(w, 4)
        r, h = divmod(k, 2)
        pltpu.async_copy(gbuf.at[b],
                         acc.at[rbuf.at[p % 3, r, pl.ds(h * 64, 64)]],
                         ss[b], add=True)

    def s_wait(w):
        b = w % 6
        pltpu.make_async_copy(gbuf.at[b], acc.at[rbuf.at[0, 0, pl.ds(0, 64)]],
                              ss[b]).wait()

    idx_load(0)
    idx_wait(0)
    idx_load(1)
    for w in range(_NWIN):
        p, k = divmod(w, 4)
        if w >= 6:
            s_wait(w - 6)
        if k == 0 and p >= 1:
            idx_wait(p)
        if k == 1 and p >= 1 and p + 1 < _NSTG:
            idx_load(p + 1)
        g_fire(w)
        if w >= 4:
            g_wait(w - 4)
            s_fire(w - 4)
    for w in range(_NWIN - 4, _NWIN):
        g_wait(w)
        s_fire(w)
    for w in range(_NWIN - 6, _NWIN):
        s_wait(w)

    plsc.subcore_barrier()

    @pl.when(s < 15)
    def _():
        pltpu.sync_copy(acc.at[pl.ds(s * 632, 632)],
                        part_hbm.at[c].at[pl.ds(s * 632, 632)])

    @pl.when(s == 15)
    def _():
        pltpu.sync_copy(acc.at[pl.ds(9480, 520)],
                        part_hbm.at[c].at[pl.ds(9480, 520)])


def _tc_out0(xp, w0, w2, b2d, rows, cols):
    def body(x_ref, w0_ref, w2_ref, b_ref, r_ref, c_ref, o_ref, m_ref):
        o_ref[...] = _dot(x_ref[...], w0_ref[...] - w2_ref[...]) + b_ref[...]
        cv = c_ref[...]
        m_ref[...] = jnp.where(r_ref[...] == cv, _ZROW, cv)

    return pl.pallas_call(
        body,
        grid=(10,),
        in_specs=[
            pl.BlockSpec((1024, _C), lambda i: (i, 0)),
            pl.BlockSpec((_C, _C), lambda i: (0, 0)),
            pl.BlockSpec((_C, _C), lambda i: (0, 0)),
            pl.BlockSpec((1, _C), lambda i: (0, 0)),
            pl.BlockSpec((256, 128), lambda i: (i, 0)),
            pl.BlockSpec((256, 128), lambda i: (i, 0)),
        ],
        out_specs=[
            pl.BlockSpec((1024, _C), lambda i: (i, 0)),
            pl.BlockSpec((256, 128), lambda i: (i, 0)),
        ],
        out_shape=[
            jax.ShapeDtypeStruct((_NP, _C), _F32),
            jax.ShapeDtypeStruct((_EP // 128, 128), jnp.int32),
        ],
    )(xp, w0, w2, b2d, rows, cols)


def _tc_scale1(degp, xp):
    def body(d_ref, x_ref, dinv_ref, xs_ref):
        deg = d_ref[0, :, 0:1] + d_ref[1, :, 0:1]
        dinv = jnp.where(deg > 0.0, lax.rsqrt(deg), 0.0)
        dinv_ref[...] = dinv
        xs_ref[...] = dinv * x_ref[...]

    return pl.pallas_call(
        body,
        grid=(10,),
        in_specs=[
            pl.BlockSpec((2, 1024, _C), lambda i: (0, i, 0)),
            pl.BlockSpec((1024, _C), lambda i: (i, 0)),
        ],
        out_specs=[
            pl.BlockSpec((1024, 1), lambda i: (i, 0)),
            pl.BlockSpec((1024, _C), lambda i: (i, 0)),
        ],
        out_shape=[
            jax.ShapeDtypeStruct((_NP, 1), _F32),
            jax.ShapeDtypeStruct((_NP, _C), _F32),
        ],
    )(degp, xp)


def _tc_comb1(part, dinv2, out0, w1):
    def body(p_ref, d_ref, o0_ref, w_ref, o1_ref, y_ref):
        u = p_ref[0] + p_ref[1]
        d = d_ref[...]
        du = d * u
        o1_ref[...] = o0_ref[...] - _dot(du, w_ref[...])
        grow = pl.program_id(0) * 1024 + lax.broadcasted_iota(
            jnp.int32, (1024, 1), 0)
        y_ref[...] = jnp.where(grow < _N, d * du, 0.0)

    return pl.pallas_call(
        body,
        grid=(10,),
        in_specs=[
            pl.BlockSpec((2, 1024, _C), lambda i: (0, i, 0)),
            pl.BlockSpec((1024, 1), lambda i: (i, 0)),
            pl.BlockSpec((1024, _C), lambda i: (i, 0)),
            pl.BlockSpec((_C, _C), lambda i: (0, 0)),
        ],
        out_specs=[
            pl.BlockSpec((1024, _C), lambda i: (i, 0)),
            pl.BlockSpec((1024, _C), lambda i: (i, 0)),
        ],
        out_shape=[
            jax.ShapeDtypeStruct((_NP, _C), _F32),
            jax.ShapeDtypeStruct((_NP, _C), _F32),
        ],
    )(part, dinv2, out0, w1)


def _tc_comb2(part, dinv2, out1, w2):
    def body(p_ref, d_ref, o1_ref, w_ref, o_ref):
        u = p_ref[0] + p_ref[1]
        du = d_ref[...] * u
        o_ref[...] = o1_ref[...] + 2.0 * _dot(du, w_ref[...])

    return pl.pallas_call(
        body,
        grid=(10,),
        in_specs=[
            pl.BlockSpec((2, 1000, _C), lambda i: (0, i, 0)),
            pl.BlockSpec((1000, 1), lambda i: (i, 0)),
            pl.BlockSpec((1000, _C), lambda i: (i, 0)),
            pl.BlockSpec((_C, _C), lambda i: (0, 0)),
        ],
        out_specs=pl.BlockSpec((1000, _C), lambda i: (i, 0)),
        out_shape=jax.ShapeDtypeStruct((_N, _C), _F32),
    )(part, dinv2, out1, w2)


def kernel(x, edge_index, weight, bias):
    xp = jnp.pad(x[0], ((0, _NP - _N), (0, 0)))
    ei = edge_index.astype(jnp.int32)
    rows = jnp.pad(ei[0], (0, _EP - _E)).reshape(_EP // 128, 128)
    cols = jnp.pad(ei[1], (0, _EP - _E)).reshape(_EP // 128, 128)
    zrow = jnp.zeros((_SL, _C), _F32)
    b2d = bias.reshape(1, _C)

    degp = _deg_kernel(rows, cols, zrow)
    out0, cmask = _tc_out0(xp, weight[0], weight[2], b2d, rows, cols)
    dinv2, xs = _tc_scale1(degp, xp)
    part1 = _spmm_kernel(xs, cmask, rows, zrow)
    out1, ytab = _tc_comb1(part1, dinv2, out0, weight[1])
    part2 = _spmm_kernel(ytab, cmask, rows, zrow)
    out = _tc_comb2(part2, dinv2, out1, weight[2])
    return out[None]


# speedup vs baseline: 1.1454x; 1.1413x over previous
import functools

import jax
import jax.numpy as jnp
from jax import lax
from jax.experimental import pallas as pl
from jax.experimental.pallas import tpu as pltpu
from jax.experimental.pallas import tpu_sc as plsc

_N = 10000
_NP = 10240
_E = 320000
_EP = 327680
_C = 128
_TRASH = _N
_NW = 32
_EPW = _EP // _NW
_SL = _NP // 16
_F32 = jnp.float32

_mesh = plsc.VectorSubcoreMesh(core_axis_name="c", subcore_axis_name="s")


def _dot(a, b):
    return lax.dot_general(
        a, b, (((1,), (0,)), ((), ())),
        precision=lax.Precision.HIGHEST, preferred_element_type=_F32)


@functools.partial(
    pl.kernel,
    out_type=jax.ShapeDtypeStruct((2, _NP, _C), _F32),
    mesh=_mesh,
    scratch_types=[
        pltpu.VMEM_SHARED((_NP, _C), _F32),
        pltpu.VMEM((128, _C), _F32),
        pltpu.VMEM((4, 128), jnp.int32),
        pltpu.VMEM((4, 128), jnp.int32),
        pltpu.VMEM((4, 128), jnp.int32),
    ],
)
def _deg_kernel(rows_hbm, cols_hbm, zrow_hbm, deg_hbm,
                acc, obuf, rbuf, cbuf, mbuf):
    c = lax.axis_index("c")
    s = lax.axis_index("s")
    wid = c * 16 + s
    ones16 = jnp.ones((16,), _F32)
    for i in range(128):
        for l in range(8):
            obuf[i, pl.ds(l * 16, 16)] = ones16
    pltpu.sync_copy(zrow_hbm, acc.at[pl.ds(s * _SL, _SL)])
    plsc.subcore_barrier()

    @pl.loop(0, _EPW // 512)
    def _(t):
        base = wid * 80 + t * 4
        pltpu.sync_copy(rows_hbm.at[pl.ds(base, 4)], rbuf)
        pltpu.sync_copy(cols_hbm.at[pl.ds(base, 4)], cbuf)
        for j in range(4):
            for l in range(8):
                rv = rbuf[j, pl.ds(l * 16, 16)]
                cv = cbuf[j, pl.ds(l * 16, 16)]
                mbuf[j, pl.ds(l * 16, 16)] = jnp.where(rv == cv, _TRASH, rv)
        for j in range(4):
            pltpu.sync_copy(obuf, acc.at[mbuf.at[j]], add=True)

    plsc.subcore_barrier()
    pltpu.sync_copy(acc.at[pl.ds(s * _SL, _SL)],
                    deg_hbm.at[c].at[pl.ds(s * _SL, _SL)])


_NWIN = 320
_NSTG = 160
_HT = 5120
_HTZ = _HT
_ZROW = 10200


@functools.partial(
    pl.kernel,
    out_type=jax.ShapeDtypeStruct((2, _N, _C), _F32),
    mesh=_mesh,
    scratch_types=[
        pltpu.VMEM_SHARED((_N, _C), _F32),
        pltpu.VMEM_SHARED((_HT + 8, _C), _F32),
        pltpu.VMEM((2, 1, 128), jnp.int32),
        pltpu.VMEM((2, 1, 128), jnp.int32),
        pltpu.VMEM((64, _C), _F32),
        pltpu.SemaphoreType.DMA,
        pltpu.SemaphoreType.DMA,
        pltpu.SemaphoreType.DMA,
    ],
)
def _spmm_kernel(table_hbm, gidx_hbm, rows_hbm, zrow_hbm, part_hbm,
                 acc, tab, cbuf, rbuf, gbuf,
                 sg, si0, si1):
    c = lax.axis_index("c")
    s = lax.axis_index("s")
    si = (si0, si1)

    pltpu.sync_copy(table_hbm.at[pl.ds(c * _HT + s * 320, 320)],
                    tab.at[pl.ds(s * 320, 320)])

    @pl.when(s == 0)
    def _():
        pltpu.sync_copy(zrow_hbm.at[pl.ds(0, 8)], tab.at[pl.ds(_HT, 8)])

    @pl.when(s < 15)
    def _():
        pltpu.sync_copy(zrow_hbm.at[pl.ds(0, 632)], acc.at[pl.ds(s * 632, 632)])

    @pl.when(s == 15)
    def _():
        pltpu.sync_copy(zrow_hbm.at[pl.ds(0, 520)], acc.at[pl.ds(9480, 520)])

    plsc.subcore_barrier()

    def idx_load(p):
        b = p % 2
        base = s * _NSTG + p
        pltpu.async_copy(gidx_hbm.at[c].at[pl.ds(base, 1)], cbuf.at[b], si[b])
        pltpu.async_copy(rows_hbm.at[pl.ds(base, 1)], rbuf.at[b], si[b])

    def idx_wait(p):
        b = p % 2
        pltpu.make_async_copy(rows_hbm.at[pl.ds(0, 1)], cbuf.at[b], si[b]).wait()
        pltpu.make_async_copy(rows_hbm.at[pl.ds(0, 1)], rbuf.at[b], si[b]).wait()

    idx_load(0)
    idx_wait(0)
    idx_load(1)
    for w in range(_NWIN):
        p, h = divmod(w, 2)
        if h == 0 and p >= 1:
            idx_wait(p)
            if p + 1 < _NSTG:
                idx_load(p + 1)
        pltpu.async_copy(tab.at[cbuf.at[p % 2, 0, pl.ds(h * 64, 64)]],
                         gbuf, sg)
        pltpu.make_async_copy(tab.at[cbuf.at[0, 0, pl.ds(0, 64)]],
                              gbuf, sg).wait()
        pltpu.sync_copy(gbuf, acc.at[rbuf.at[p % 2, 0, pl.ds(h * 64, 64)]],
                        add=True)

    plsc.subcore_barrier()

    @pl.when(s < 15)
    def _():
        pltpu.sync_copy(acc.at[pl.ds(s * 632, 632)],
                        part_hbm.at[c].at[pl.ds(s * 632, 632)])

    @pl.when(s == 15)
    def _():
        pltpu.sync_copy(acc.at[pl.ds(9480, 520)],
                        part_hbm.at[c].at[pl.ds(9480, 520)])


def _tc_out0(xp, w0, w2, b2d, rows, cols):
    def body(x_ref, w0_ref, w2_ref, b_ref, r_ref, c_ref, o_ref, m_ref):
        o_ref[...] = _dot(x_ref[...], w0_ref[...] - w2_ref[...]) + b_ref[...]
        cv = c_ref[...]
        rv = r_ref[...]
        for i in range(2):
            loc = (cv >= i * _HT) & (cv < (i + 1) * _HT) & (rv != cv)
            m_ref[i] = jnp.where(loc, cv - i * _HT, _HTZ)

    return pl.pallas_call(
        body,
        grid=(10,),
        in_specs=[
            pl.BlockSpec((1024, _C), lambda i: (i, 0)),
            pl.BlockSpec((_C, _C), lambda i: (0, 0)),
            pl.BlockSpec((_C, _C), lambda i: (0, 0)),
            pl.BlockSpec((1, _C), lambda i: (0, 0)),
            pl.BlockSpec((256, 128), lambda i: (i, 0)),
            pl.BlockSpec((256, 128), lambda i: (i, 0)),
        ],
        out_specs=[
            pl.BlockSpec((1024, _C), lambda i: (i, 0)),
            pl.BlockSpec((2, 256, 128), lambda i: (0, i, 0)),
        ],
        out_shape=[
            jax.ShapeDtypeStruct((_NP, _C), _F32),
            jax.ShapeDtypeStruct((2, _EP // 128, 128), jnp.int32),
        ],
    )(xp, w0, w2, b2d, rows, cols)


def _tc_scale1(degp, xp):
    def body(d_ref, x_ref, dinv_ref, xs_ref):
        deg = d_ref[0, :, 0:1] + d_ref[1, :, 0:1]
        dinv = jnp.where(deg > 0.0, lax.rsqrt(deg), 0.0)
        dinv_ref[...] = dinv
        xs_ref[...] = dinv * x_ref[...]

    return pl.pallas_call(
        body,
        grid=(10,),
        in_specs=[
            pl.BlockSpec((2, 1024, _C), lambda i: (0, i, 0)),
            pl.BlockSpec((1024, _C), lambda i: (i, 0)),
        ],
        out_specs=[
            pl.BlockSpec((1024, 1), lambda i: (i, 0)),
            pl.BlockSpec((1024, _C), lambda i: (i, 0)),
        ],
        out_shape=[
            jax.ShapeDtypeStruct((_NP, 1), _F32),
            jax.ShapeDtypeStruct((_NP, _C), _F32),
        ],
    )(degp, xp)


def _tc_comb1(part, dinv2, out0, w1):
    def body(p_ref, d_ref, o0_ref, w_ref, o1_ref, y_ref):
        u = p_ref[0] + p_ref[1]
        d = d_ref[...]
        du = d * u
        o1_ref[...] = o0_ref[...] - _dot(du, w_ref[...])
        grow = pl.program_id(0) * 1024 + lax.broadcasted_iota(
            jnp.int32, (1024, 1), 0)
        y_ref[...] = jnp.where(grow < _N, d * du, 0.0)

    return pl.pallas_call(
        body,
        grid=(10,),
        in_specs=[
            pl.BlockSpec((2, 1024, _C), lambda i: (0, i, 0)),
            pl.BlockSpec((1024, 1), lambda i: (i, 0)),
            pl.BlockSpec((1024, _C), lambda i: (i, 0)),
            pl.BlockSpec((_C, _C), lambda i: (0, 0)),
        ],
        out_specs=[
            pl.BlockSpec((1024, _C), lambda i: (i, 0)),
            pl.BlockSpec((1024, _C), lambda i: (i, 0)),
        ],
        out_shape=[
            jax.ShapeDtypeStruct((_NP, _C), _F32),
            jax.ShapeDtypeStruct((_NP, _C), _F32),
        ],
    )(part, dinv2, out0, w1)


def _tc_comb2(part, dinv2, out1, w2):
    def body(p_ref, d_ref, o1_ref, w_ref, o_ref):
        u = p_ref[0] + p_ref[1]
        du = d_ref[...] * u
        o_ref[...] = o1_ref[...] + 2.0 * _dot(du, w_ref[...])

    return pl.pallas_call(
        body,
        grid=(10,),
        in_specs=[
            pl.BlockSpec((2, 1000, _C), lambda i: (0, i, 0)),
            pl.BlockSpec((1000, 1), lambda i: (i, 0)),
            pl.BlockSpec((1000, _C), lambda i: (i, 0)),
            pl.BlockSpec((_C, _C), lambda i: (0, 0)),
        ],
        out_specs=pl.BlockSpec((1000, _C), lambda i: (i, 0)),
        out_shape=jax.ShapeDtypeStruct((_N, _C), _F32),
    )(part, dinv2, out1, w2)


def kernel(x, edge_index, weight, bias):
    xp = jnp.pad(x[0], ((0, _NP - _N), (0, 0)))
    ei = edge_index.astype(jnp.int32)
    rows = jnp.pad(ei[0], (0, _EP - _E)).reshape(_EP // 128, 128)
    cols = jnp.pad(ei[1], (0, _EP - _E)).reshape(_EP // 128, 128)
    zrow = jnp.zeros((_SL, _C), _F32)
    b2d = bias.reshape(1, _C)

    degp = _deg_kernel(rows, cols, zrow)
    out0, cmask = _tc_out0(xp, weight[0], weight[2], b2d, rows, cols)
    dinv2, xs = _tc_scale1(degp, xp)
    part1 = _spmm_kernel(xs, cmask, rows, zrow)
    out1, ytab = _tc_comb1(part1, dinv2, out0, weight[1])
    part2 = _spmm_kernel(ytab, cmask, rows, zrow)
    out = _tc_comb2(part2, dinv2, out1, weight[2])
    return out[None]
